# baseline (speedup 1.0000x reference)
import functools

import jax
import jax.numpy as jnp
from jax import lax
from jax.experimental import pallas as pl
from jax.experimental.pallas import tpu as pltpu
from jax.experimental.pallas import tpu_sc as plsc

_NC = 2
_NS = 16
_NW = _NC * _NS
_L = 16

_D = 64
_NH = 50
_B = 4096
_BW = _B // _NW
_DC = _D // _L


def _sc_body(user_hbm, memt_hbm, ut_hbm, it_hbm, w_hbm, bias_hbm, out_hbm,
             uidx_v, midx_v, ue_v, rows_v, acc_v, out_v, w_v, bias_v,
             sem_u, sem_g0, sem_g1):
    wid = lax.axis_index("s") * _NC + lax.axis_index("c")
    base = wid * _BW

    pltpu.sync_copy(user_hbm.at[pl.ds(base, _BW)], uidx_v)
    pltpu.sync_copy(memt_hbm.at[:, pl.ds(base, _BW)], midx_v)
    pltpu.sync_copy(w_hbm, w_v)
    pltpu.sync_copy(bias_hbm, bias_v)

    ue_cp = pltpu.async_copy(ut_hbm.at[uidx_v], ue_v, sem_u)

    sems = (sem_g0, sem_g1)
    gathers = [None, None]

    def start_gather(n):
        p = n & 1
        gathers[p] = pltpu.async_copy(
            it_hbm.at[midx_v.at[n]], rows_v.at[p], sems[p])

    start_gather(0)
    start_gather(1)
    for n in range(_NH):
        p = n & 1
        gathers[p].wait()
        wv = w_v[n, :]

        def acc_body(b, _, p=p, wv=wv, first=(n == 0)):
            for d in range(_DC):
                r = rows_v[p, b, pl.ds(d * _L, _L)]
                if first:
                    acc_v[b, pl.ds(d * _L, _L)] = wv * r
                else:
                    acc_v[b, pl.ds(d * _L, _L)] += wv * r
            return 0

        lax.fori_loop(0, _BW, acc_body, 0)
        if n + 2 < _NH:
            start_gather(n + 2)

    ue_cp.wait()
    bias = bias_v[:]

    def comb_body(b, _):
        for d in range(_DC):
            u = ue_v[b, pl.ds(d * _L, _L)]
            a = acc_v[b, pl.ds(d * _L, _L)] + bias
            out_v[b, pl.ds(d * _L, _L)] = u
            out_v[b, pl.ds(_D + d * _L, _L)] = u * a
            out_v[b, pl.ds(2 * _D + d * _L, _L)] = a
        return 0

    lax.fori_loop(0, _BW, comb_body, 0)
    pltpu.sync_copy(out_v, out_hbm.at[pl.ds(base, _BW), :])


@jax.jit
def kernel(user, memory, user_table, item_table, conv_w, conv_b):
    user_idx = user.reshape(_B).astype(jnp.int32)
    mem_t = memory.astype(jnp.int32).T
    w2 = jnp.broadcast_to(conv_w.reshape(_NH, 1), (_NH, _L)).astype(jnp.float32)
    bias16 = jnp.broadcast_to(conv_b.reshape(1), (_L,)).astype(jnp.float32)

    mesh = plsc.VectorSubcoreMesh(core_axis_name="c", subcore_axis_name="s",
                                  num_cores=_NC, num_subcores=_NS)
    run = pl.kernel(
        _sc_body,
        out_type=jax.ShapeDtypeStruct((_B, 3 * _D), jnp.float32),
        mesh=mesh,
        scratch_types=[
            pltpu.VMEM((_BW,), jnp.int32),
            pltpu.VMEM((_NH, _BW), jnp.int32),
            pltpu.VMEM((_BW, _D), jnp.float32),
            pltpu.VMEM((2, _BW, _D), jnp.float32),
            pltpu.VMEM((_BW, _D), jnp.float32),
            pltpu.VMEM((_BW, 3 * _D), jnp.float32),
            pltpu.VMEM((_NH, _L), jnp.float32),
            pltpu.VMEM((_L,), jnp.float32),
            pltpu.SemaphoreType.DMA,
            pltpu.SemaphoreType.DMA,
            pltpu.SemaphoreType.DMA,
        ],
        compiler_params=pltpu.CompilerParams(use_tc_tiling_on_sc=False),
    )
    return run(user_idx, mem_t, user_table, item_table, w2, bias16)


# speedup vs baseline: 1.0712x; 1.0712x over previous
import functools

import jax
import jax.numpy as jnp
from jax import lax
from jax.experimental import pallas as pl
from jax.experimental.pallas import tpu as pltpu
from jax.experimental.pallas import tpu_sc as plsc

_NC = 2
_NS = 16
_NW = _NC * _NS
_L = 16

_D = 64
_DP = 128
_NH = 50
_B = 4096
_BW = _B // _NW
_DC = _D // _L


def _sc_body(user_hbm, memt_hbm, ut_hbm, it_hbm, w_hbm, bias_hbm, out_hbm,
             uidx_v, midx_v, ue_v, rows_v, acc_v, out_v, w_v, bias_v,
             sem_u, sem_g0, sem_g1):
    wid = lax.axis_index("s") * _NC + lax.axis_index("c")
    base = wid * _BW

    pltpu.sync_copy(user_hbm.at[pl.ds(base, _BW)], uidx_v)
    pltpu.sync_copy(memt_hbm.at[:, pl.ds(base, _BW)], midx_v)
    pltpu.sync_copy(w_hbm, w_v)
    pltpu.sync_copy(bias_hbm, bias_v)

    ue_cp = pltpu.async_copy(ut_hbm.at[uidx_v], ue_v, sem_u)

    sems = (sem_g0, sem_g1)
    gathers = [None, None]

    def start_gather(n):
        p = n & 1
        gathers[p] = pltpu.async_copy(
            it_hbm.at[midx_v.at[n]], rows_v.at[p], sems[p])

    start_gather(0)
    start_gather(1)
    for n in range(_NH):
        p = n & 1
        gathers[p].wait()
        wv = w_v[n, :]

        def acc_body(b, _, p=p, wv=wv, first=(n == 0)):
            for d in range(_DC):
                r = rows_v[p, b, pl.ds(d * _L, _L)]
                if first:
                    acc_v[b, pl.ds(d * _L, _L)] = wv * r
                else:
                    acc_v[b, pl.ds(d * _L, _L)] += wv * r
            return 0

        lax.fori_loop(0, _BW, acc_body, 0)
        if n + 2 < _NH:
            start_gather(n + 2)

    ue_cp.wait()
    bias = bias_v[:]

    def comb_body(b, _):
        for d in range(_DC):
            u = ue_v[b, pl.ds(d * _L, _L)]
            a = acc_v[b, pl.ds(d * _L, _L)] + bias
            out_v[b, pl.ds(d * _L, _L)] = u
            out_v[b, pl.ds(_D + d * _L, _L)] = u * a
            out_v[b, pl.ds(2 * _D + d * _L, _L)] = a
        return 0

    lax.fori_loop(0, _BW, comb_body, 0)
    pltpu.sync_copy(out_v, out_hbm.at[pl.ds(base, _BW), :])


@jax.jit
def kernel(user, memory, user_table, item_table, conv_w, conv_b):
    user_idx = user.reshape(_B).astype(jnp.int32)
    mem_t = memory.astype(jnp.int32).T
    w2 = jnp.broadcast_to(conv_w.reshape(_NH, 1), (_NH, _L)).astype(jnp.float32)
    bias16 = jnp.broadcast_to(conv_b.reshape(1), (_L,)).astype(jnp.float32)
    it128 = jnp.pad(item_table, ((0, 0), (0, _DP - _D)))
    ut128 = jnp.pad(user_table, ((0, 0), (0, _DP - _D)))

    mesh = plsc.VectorSubcoreMesh(core_axis_name="c", subcore_axis_name="s",
                                  num_cores=_NC, num_subcores=_NS)
    run = pl.kernel(
        _sc_body,
        out_type=jax.ShapeDtypeStruct((_B, 3 * _D), jnp.float32),
        mesh=mesh,
        scratch_types=[
            pltpu.VMEM((_BW,), jnp.int32),
            pltpu.VMEM((_NH, _BW), jnp.int32),
            pltpu.VMEM((_BW, _DP), jnp.float32),
            pltpu.VMEM((2, _BW, _DP), jnp.float32),
            pltpu.VMEM((_BW, _D), jnp.float32),
            pltpu.VMEM((_BW, 3 * _D), jnp.float32),
            pltpu.VMEM((_NH, _L), jnp.float32),
            pltpu.VMEM((_L,), jnp.float32),
            pltpu.SemaphoreType.DMA,
            pltpu.SemaphoreType.DMA,
            pltpu.SemaphoreType.DMA,
        ],
    )
    return run(user_idx, mem_t, ut128, it128, w2, bias16)


# speedup vs baseline: 1.1801x; 1.1016x over previous
import functools

import jax
import jax.numpy as jnp
from jax import lax
from jax.experimental import pallas as pl
from jax.experimental.pallas import tpu as pltpu
from jax.experimental.pallas import tpu_sc as plsc

_NC = 2
_NS = 16
_NW = _NC * _NS
_L = 16

_D = 64
_DP = 128
_NH = 50
_B = 4096
_BW = _B // _NW
_DC = _D // _L

_TC_BLK = 2048


def _xpose_body(int_ref, out_ref):
    x = int_ref[...]
    xt = jnp.swapaxes(x, 0, 1)
    out_ref[...] = jnp.concatenate(
        [xt, jnp.zeros((_TC_BLK, _DP - _D), jnp.float32)], axis=1)


def _widen(table_t, rows):
    nblk = (rows + _TC_BLK - 1) // _TC_BLK
    return pl.pallas_call(
        _xpose_body,
        grid=(nblk,),
        in_specs=[pl.BlockSpec((_D, _TC_BLK), lambda i: (0, i))],
        out_specs=pl.BlockSpec((_TC_BLK, _DP), lambda i: (i, 0)),
        out_shape=jax.ShapeDtypeStruct((nblk * _TC_BLK, _DP), jnp.float32),
    )(table_t)


def _sc_body(user_hbm, memt_hbm, ut_hbm, it_hbm, w_hbm, bias_hbm, out_hbm,
             uidx_v, midx_v, ue_v, rows_v, acc_v, out_v, w_v, bias_v,
             sem_u, sem_g0, sem_g1):
    wid = lax.axis_index("s") * _NC + lax.axis_index("c")
    base = wid * _BW

    pltpu.sync_copy(user_hbm.at[pl.ds(base, _BW)], uidx_v)
    pltpu.sync_copy(memt_hbm.at[:, pl.ds(base, _BW)], midx_v)
    pltpu.sync_copy(w_hbm, w_v)
    pltpu.sync_copy(bias_hbm, bias_v)

    ue_cp = pltpu.async_copy(ut_hbm.at[uidx_v], ue_v, sem_u)

    sems = (sem_g0, sem_g1)
    gathers = [None, None]

    def start_gather(n):
        p = n & 1
        gathers[p] = pltpu.async_copy(
            it_hbm.at[midx_v.at[n]], rows_v.at[p], sems[p])

    start_gather(0)
    start_gather(1)
    for n in range(_NH):
        p = n & 1
        gathers[p].wait()
        wv = w_v[n, :]

        def acc_body(b, _, p=p, wv=wv, first=(n == 0)):
            for d in range(_DC):
                r = rows_v[p, b, pl.ds(d * _L, _L)]
                if first:
                    acc_v[b, pl.ds(d * _L, _L)] = wv * r
                else:
                    acc_v[b, pl.ds(d * _L, _L)] += wv * r
            return 0

        lax.fori_loop(0, _BW, acc_body, 0)
        if n + 2 < _NH:
            start_gather(n + 2)

    ue_cp.wait()
    bias = bias_v[:]

    def comb_body(b, _):
        for d in range(_DC):
            u = ue_v[b, pl.ds(d * _L, _L)]
            a = acc_v[b, pl.ds(d * _L, _L)] + bias
            out_v[b, pl.ds(d * _L, _L)] = u
            out_v[b, pl.ds(_D + d * _L, _L)] = u * a
            out_v[b, pl.ds(2 * _D + d * _L, _L)] = a
        return 0

    lax.fori_loop(0, _BW, comb_body, 0)
    pltpu.sync_copy(out_v, out_hbm.at[pl.ds(base, _BW), :])


@jax.jit
def kernel(user, memory, user_table, item_table, conv_w, conv_b):
    user_idx = user.reshape(_B).astype(jnp.int32)
    mem_t = memory.astype(jnp.int32).T
    w2 = jnp.broadcast_to(conv_w.reshape(_NH, 1), (_NH, _L)).astype(jnp.float32)
    bias16 = jnp.broadcast_to(conv_b.reshape(1), (_L,)).astype(jnp.float32)
    it128 = _widen(item_table.T, item_table.shape[0])
    ut128 = _widen(user_table.T, user_table.shape[0])

    mesh = plsc.VectorSubcoreMesh(core_axis_name="c", subcore_axis_name="s",
                                  num_cores=_NC, num_subcores=_NS)
    run = pl.kernel(
        _sc_body,
        out_type=jax.ShapeDtypeStruct((_B, 3 * _D), jnp.float32),
        mesh=mesh,
        scratch_types=[
            pltpu.VMEM((_BW,), jnp.int32),
            pltpu.VMEM((_NH, _BW), jnp.int32),
            pltpu.VMEM((_BW, _DP), jnp.float32),
            pltpu.VMEM((2, _BW, _DP), jnp.float32),
            pltpu.VMEM((_BW, _D), jnp.float32),
            pltpu.VMEM((_BW, 3 * _D), jnp.float32),
            pltpu.VMEM((_NH, _L), jnp.float32),
            pltpu.VMEM((_L,), jnp.float32),
            pltpu.SemaphoreType.DMA,
            pltpu.SemaphoreType.DMA,
            pltpu.SemaphoreType.DMA,
        ],
    )
    return run(user_idx, mem_t, ut128, it128, w2, bias16)


# speedup vs baseline: 1.8599x; 1.5761x over previous
import functools

import jax
import jax.numpy as jnp
from jax import lax
from jax.experimental import pallas as pl
from jax.experimental.pallas import tpu as pltpu
from jax.experimental.pallas import tpu_sc as plsc

_NC = 2
_NS = 16
_NW = _NC * _NS
_L = 16

_D = 64
_DP = 128
_NH = 50
_B = 4096
_BW = _B // _NW
_DC = _D // _L

_TC_BLK = 8192


def _xpose_body(int_ref, out_ref):
    x = int_ref[...]
    xt = jnp.swapaxes(x, 0, 1)
    out_ref[...] = jnp.concatenate(
        [xt, jnp.zeros((_TC_BLK, _DP - _D), jnp.float32)], axis=1)


def _widen(table_t, rows):
    nblk = (rows + _TC_BLK - 1) // _TC_BLK
    return pl.pallas_call(
        _xpose_body,
        grid=(nblk,),
        in_specs=[pl.BlockSpec((_D, _TC_BLK), lambda i: (0, i))],
        out_specs=pl.BlockSpec((_TC_BLK, _DP), lambda i: (i, 0)),
        out_shape=jax.ShapeDtypeStruct((nblk * _TC_BLK, _DP), jnp.float32),
    )(table_t)


def _sc_body(user_hbm, memt_hbm, ut_hbm, it_hbm, w_hbm, bias_hbm, out_hbm,
             uidx_v, midx_v, ue_v, rows_v, acc_v, out_v, w_v, bias_v,
             sem_u, sem_g0, sem_g1):
    wid = lax.axis_index("s") * _NC + lax.axis_index("c")
    base = wid * _BW

    pltpu.sync_copy(user_hbm.at[pl.ds(base, _BW)], uidx_v)
    pltpu.sync_copy(memt_hbm.at[:, pl.ds(base, _BW)], midx_v)
    pltpu.sync_copy(w_hbm, w_v)
    pltpu.sync_copy(bias_hbm, bias_v)

    ue_cp = pltpu.async_copy(ut_hbm.at[uidx_v], ue_v, sem_u)

    sems = (sem_g0, sem_g1)
    gathers = [None, None]

    def start_gather(n):
        p = n & 1
        gathers[p] = pltpu.async_copy(
            it_hbm.at[midx_v.at[n]], rows_v.at[p], sems[p])

    start_gather(0)
    start_gather(1)
    for n in range(_NH):
        p = n & 1
        gathers[p].wait()
        wv = w_v[n, :]

        def acc_body(b, _, p=p, wv=wv, first=(n == 0)):
            for d in range(_DC):
                r = rows_v[p, b, pl.ds(d * _L, _L)]
                if first:
                    acc_v[b, pl.ds(d * _L, _L)] = wv * r
                else:
                    acc_v[b, pl.ds(d * _L, _L)] += wv * r
            return 0

        lax.fori_loop(0, _BW, acc_body, 0)
        if n + 2 < _NH:
            start_gather(n + 2)

    ue_cp.wait()
    bias = bias_v[:]

    def comb_body(b, _):
        for d in range(_DC):
            u = ue_v[b, pl.ds(d * _L, _L)]
            a = acc_v[b, pl.ds(d * _L, _L)] + bias
            out_v[b, pl.ds(d * _L, _L)] = u
            out_v[b, pl.ds(_D + d * _L, _L)] = u * a
            out_v[b, pl.ds(2 * _D + d * _L, _L)] = a
        return 0

    lax.fori_loop(0, _BW, comb_body, 0)
    pltpu.sync_copy(out_v, out_hbm.at[pl.ds(base, _BW), :])


@jax.jit
def kernel(user, memory, user_table, item_table, conv_w, conv_b):
    user_idx = user.reshape(_B).astype(jnp.int32)
    mem_t = memory.astype(jnp.int32).T
    w2 = jnp.broadcast_to(conv_w.reshape(_NH, 1), (_NH, _L)).astype(jnp.float32)
    bias16 = jnp.broadcast_to(conv_b.reshape(1), (_L,)).astype(jnp.float32)
    it128 = _widen(item_table.T, item_table.shape[0])
    ut128 = _widen(user_table.T, user_table.shape[0])

    mesh = plsc.VectorSubcoreMesh(core_axis_name="c", subcore_axis_name="s",
                                  num_cores=_NC, num_subcores=_NS)
    run = pl.kernel(
        _sc_body,
        out_type=jax.ShapeDtypeStruct((_B, 3 * _D), jnp.float32),
        mesh=mesh,
        scratch_types=[
            pltpu.VMEM((_BW,), jnp.int32),
            pltpu.VMEM((_NH, _BW), jnp.int32),
            pltpu.VMEM((_BW, _DP), jnp.float32),
            pltpu.VMEM((2, _BW, _DP), jnp.float32),
            pltpu.VMEM((_BW, _D), jnp.float32),
            pltpu.VMEM((_BW, 3 * _D), jnp.float32),
            pltpu.VMEM((_NH, _L), jnp.float32),
            pltpu.VMEM((_L,), jnp.float32),
            pltpu.SemaphoreType.DMA,
            pltpu.SemaphoreType.DMA,
            pltpu.SemaphoreType.DMA,
        ],
    )
    return run(user_idx, mem_t, ut128, it128, w2, bias16)


# speedup vs baseline: 1.9518x; 1.0494x over previous
import functools

import jax
import jax.numpy as jnp
from jax import lax
from jax.experimental import pallas as pl
from jax.experimental.pallas import tpu as pltpu
from jax.experimental.pallas import tpu_sc as plsc

_NC = 2
_NS = 16
_NW = _NC * _NS
_L = 16

_D = 64
_DP = 128
_NH = 50
_B = 4096
_BW = _B // _NW
_DC = _D // _L

_TC_BLK = 16384


def _xpose_body(int_ref, out_ref):
    x = int_ref[...]
    xt = jnp.swapaxes(x, 0, 1)
    out_ref[...] = jnp.concatenate(
        [xt, jnp.zeros((_TC_BLK, _DP - _D), jnp.float32)], axis=1)


def _widen(table_t, rows):
    nblk = (rows + _TC_BLK - 1) // _TC_BLK
    return pl.pallas_call(
        _xpose_body,
        grid=(nblk,),
        in_specs=[pl.BlockSpec((_D, _TC_BLK), lambda i: (0, i))],
        out_specs=pl.BlockSpec((_TC_BLK, _DP), lambda i: (i, 0)),
        out_shape=jax.ShapeDtypeStruct((nblk * _TC_BLK, _DP), jnp.float32),
    )(table_t)


def _sc_body(user_hbm, memt_hbm, ut_hbm, it_hbm, w_hbm, bias_hbm, out_hbm,
             uidx_v, midx_v, ue_v, rows_v, acc_v, out_v, w_v, bias_v,
             sem_u, sem_g0, sem_g1):
    wid = lax.axis_index("s") * _NC + lax.axis_index("c")
    base = wid * _BW

    pltpu.sync_copy(user_hbm.at[pl.ds(base, _BW)], uidx_v)
    pltpu.sync_copy(memt_hbm.at[:, pl.ds(base, _BW)], midx_v)
    pltpu.sync_copy(w_hbm, w_v)
    pltpu.sync_copy(bias_hbm, bias_v)

    ue_cp = pltpu.async_copy(ut_hbm.at[uidx_v], ue_v, sem_u)

    sems = (sem_g0, sem_g1)
    gathers = [None, None]

    def start_gather(n):
        p = n & 1
        gathers[p] = pltpu.async_copy(
            it_hbm.at[midx_v.at[n]], rows_v.at[p], sems[p])

    start_gather(0)
    start_gather(1)
    for n in range(_NH):
        p = n & 1
        gathers[p].wait()
        wv = w_v[n, :]

        def acc_body(b, _, p=p, wv=wv, first=(n == 0)):
            for d in range(_DC):
                r = rows_v[p, b, pl.ds(d * _L, _L)]
                if first:
                    acc_v[b, pl.ds(d * _L, _L)] = wv * r
                else:
                    acc_v[b, pl.ds(d * _L, _L)] += wv * r
            return 0

        lax.fori_loop(0, _BW, acc_body, 0)
        if n + 2 < _NH:
            start_gather(n + 2)

    ue_cp.wait()
    bias = bias_v[:]

    def comb_body(b, _):
        for d in range(_DC):
            u = ue_v[b, pl.ds(d * _L, _L)]
            a = acc_v[b, pl.ds(d * _L, _L)] + bias
            out_v[b, pl.ds(d * _L, _L)] = u
            out_v[b, pl.ds(_D + d * _L, _L)] = u * a
            out_v[b, pl.ds(2 * _D + d * _L, _L)] = a
        return 0

    lax.fori_loop(0, _BW, comb_body, 0)
    pltpu.sync_copy(out_v, out_hbm.at[pl.ds(base, _BW), :])


@jax.jit
def kernel(user, memory, user_table, item_table, conv_w, conv_b):
    user_idx = user.reshape(_B).astype(jnp.int32)
    mem_t = memory.astype(jnp.int32).T
    w2 = jnp.broadcast_to(conv_w.reshape(_NH, 1), (_NH, _L)).astype(jnp.float32)
    bias16 = jnp.broadcast_to(conv_b.reshape(1), (_L,)).astype(jnp.float32)
    it128 = _widen(item_table.T, item_table.shape[0])
    ut128 = _widen(user_table.T, user_table.shape[0])

    mesh = plsc.VectorSubcoreMesh(core_axis_name="c", subcore_axis_name="s",
                                  num_cores=_NC, num_subcores=_NS)
    run = pl.kernel(
        _sc_body,
        out_type=jax.ShapeDtypeStruct((_B, 3 * _D), jnp.float32),
        mesh=mesh,
        scratch_types=[
            pltpu.VMEM((_BW,), jnp.int32),
            pltpu.VMEM((_NH, _BW), jnp.int32),
            pltpu.VMEM((_BW, _DP), jnp.float32),
            pltpu.VMEM((2, _BW, _DP), jnp.float32),
            pltpu.VMEM((_BW, _D), jnp.float32),
            pltpu.VMEM((_BW, 3 * _D), jnp.float32),
            pltpu.VMEM((_NH, _L), jnp.float32),
            pltpu.VMEM((_L,), jnp.float32),
            pltpu.SemaphoreType.DMA,
            pltpu.SemaphoreType.DMA,
            pltpu.SemaphoreType.DMA,
        ],
    )
    return run(user_idx, mem_t, ut128, it128, w2, bias16)


# speedup vs baseline: 1.9765x; 1.0127x over previous
import functools

import jax
import jax.numpy as jnp
from jax import lax
from jax.experimental import pallas as pl
from jax.experimental.pallas import tpu as pltpu
from jax.experimental.pallas import tpu_sc as plsc

_NC = 2
_NS = 16
_NW = _NC * _NS
_L = 16

_D = 64
_DP = 128
_NH = 50
_B = 4096
_BW = _B // _NW
_DC = _D // _L

_TC_BLK = 32768


def _xpose_body(int_ref, out_ref):
    x = int_ref[...]
    xt = jnp.swapaxes(x, 0, 1)
    out_ref[...] = jnp.concatenate(
        [xt, jnp.zeros((_TC_BLK, _DP - _D), jnp.float32)], axis=1)


def _widen(table_t, rows):
    nblk = (rows + _TC_BLK - 1) // _TC_BLK
    return pl.pallas_call(
        _xpose_body,
        grid=(nblk,),
        in_specs=[pl.BlockSpec((_D, _TC_BLK), lambda i: (0, i))],
        out_specs=pl.BlockSpec((_TC_BLK, _DP), lambda i: (i, 0)),
        out_shape=jax.ShapeDtypeStruct((nblk * _TC_BLK, _DP), jnp.float32),
    )(table_t)


def _sc_body(user_hbm, memt_hbm, ut_hbm, it_hbm, w_hbm, bias_hbm, out_hbm,
             uidx_v, midx_v, ue_v, rows_v, acc_v, out_v, w_v, bias_v,
             sem_u, sem_g0, sem_g1):
    wid = lax.axis_index("s") * _NC + lax.axis_index("c")
    base = wid * _BW

    pltpu.sync_copy(user_hbm.at[pl.ds(base, _BW)], uidx_v)
    pltpu.sync_copy(memt_hbm.at[:, pl.ds(base, _BW)], midx_v)
    pltpu.sync_copy(w_hbm, w_v)
    pltpu.sync_copy(bias_hbm, bias_v)

    ue_cp = pltpu.async_copy(ut_hbm.at[uidx_v], ue_v, sem_u)

    sems = (sem_g0, sem_g1)
    gathers = [None, None]

    def start_gather(n):
        p = n & 1
        gathers[p] = pltpu.async_copy(
            it_hbm.at[midx_v.at[n]], rows_v.at[p], sems[p])

    start_gather(0)
    start_gather(1)
    for n in range(_NH):
        p = n & 1
        gathers[p].wait()
        wv = w_v[n, :]

        def acc_body(b, _, p=p, wv=wv, first=(n == 0)):
            for d in range(_DC):
                r = rows_v[p, b, pl.ds(d * _L, _L)]
                if first:
                    acc_v[b, pl.ds(d * _L, _L)] = wv * r
                else:
                    acc_v[b, pl.ds(d * _L, _L)] += wv * r
            return 0

        lax.fori_loop(0, _BW, acc_body, 0)
        if n + 2 < _NH:
            start_gather(n + 2)

    ue_cp.wait()
    bias = bias_v[:]

    def comb_body(b, _):
        for d in range(_DC):
            u = ue_v[b, pl.ds(d * _L, _L)]
            a = acc_v[b, pl.ds(d * _L, _L)] + bias
            out_v[b, pl.ds(d * _L, _L)] = u
            out_v[b, pl.ds(_D + d * _L, _L)] = u * a
            out_v[b, pl.ds(2 * _D + d * _L, _L)] = a
        return 0

    lax.fori_loop(0, _BW, comb_body, 0)
    pltpu.sync_copy(out_v, out_hbm.at[pl.ds(base, _BW), :])


@jax.jit
def kernel(user, memory, user_table, item_table, conv_w, conv_b):
    user_idx = user.reshape(_B).astype(jnp.int32)
    mem_t = memory.astype(jnp.int32).T
    w2 = jnp.broadcast_to(conv_w.reshape(_NH, 1), (_NH, _L)).astype(jnp.float32)
    bias16 = jnp.broadcast_to(conv_b.reshape(1), (_L,)).astype(jnp.float32)
    it128 = _widen(item_table.T, item_table.shape[0])
    ut128 = _widen(user_table.T, user_table.shape[0])

    mesh = plsc.VectorSubcoreMesh(core_axis_name="c", subcore_axis_name="s",
                                  num_cores=_NC, num_subcores=_NS)
    run = pl.kernel(
        _sc_body,
        out_type=jax.ShapeDtypeStruct((_B, 3 * _D), jnp.float32),
        mesh=mesh,
        scratch_types=[
            pltpu.VMEM((_BW,), jnp.int32),
            pltpu.VMEM((_NH, _BW), jnp.int32),
            pltpu.VMEM((_BW, _DP), jnp.float32),
            pltpu.VMEM((2, _BW, _DP), jnp.float32),
            pltpu.VMEM((_BW, _D), jnp.float32),
            pltpu.VMEM((_BW, 3 * _D), jnp.float32),
            pltpu.VMEM((_NH, _L), jnp.float32),
            pltpu.VMEM((_L,), jnp.float32),
            pltpu.SemaphoreType.DMA,
            pltpu.SemaphoreType.DMA,
            pltpu.SemaphoreType.DMA,
        ],
    )
    return run(user_idx, mem_t, ut128, it128, w2, bias16)
